# baseline (speedup 1.0000x reference)
import functools
import jax
import jax.numpy as jnp
from jax import lax
from jax.experimental import pallas as pl
from jax.experimental.pallas import tpu as pltpu
from jax.experimental.pallas import tpu_sc as plsc

H = 32
LW = 128


def _embed_body(x_ref, w_ref, b_ref, o_ref):
    h = jnp.maximum(
        jnp.dot(x_ref[:], w_ref[:], preferred_element_type=jnp.float32)
        + b_ref[:], 0.0)
    o_ref[:] = jnp.concatenate(
        [h, jnp.zeros((h.shape[0], LW - H), jnp.float32)], axis=1)


def _embed(x, w, b):
    n = x.shape[0]
    return pl.pallas_call(
        _embed_body,
        out_shape=jax.ShapeDtypeStruct((n, LW), jnp.float32),
    )(x, w, b.reshape(1, H))


def _msg_body(ea_ref, hs_ref, w1_ref, b1_ref, w2_ref, b2_ref, o_ref):
    ev = jnp.maximum(
        jnp.dot(ea_ref[:], w1_ref[:], preferred_element_type=jnp.float32)
        + b1_ref[:], 0.0)
    wt = jnp.dot(ev, w2_ref[:], preferred_element_type=jnp.float32) + b2_ref[:]
    hs = hs_ref[:]
    acc = hs[:, 0:1] * wt[:, 0:H]
    for i in range(1, H):
        acc = acc + hs[:, i:i + 1] * wt[:, i * H:(i + 1) * H]
    t = acc.shape[0]
    o_ref[:] = jnp.concatenate(
        [acc, jnp.ones((t, 1), jnp.float32),
         jnp.zeros((t, LW - H - 1), jnp.float32)], axis=1)


def _msg(ea, hs, w1, b1, w2, b2):
    e = ea.shape[0]
    de = ea.shape[1]
    t = 640
    grid = (e // t,)
    return pl.pallas_call(
        _msg_body,
        grid=grid,
        in_specs=[
            pl.BlockSpec((t, de), lambda i: (i, 0)),
            pl.BlockSpec((t, LW), lambda i: (i, 0)),
            pl.BlockSpec((de, H), lambda i: (0, 0)),
            pl.BlockSpec((1, H), lambda i: (0, 0)),
            pl.BlockSpec((H, H * H), lambda i: (0, 0)),
            pl.BlockSpec((1, H * H), lambda i: (0, 0)),
        ],
        out_specs=pl.BlockSpec((t, LW), lambda i: (i, 0)),
        out_shape=jax.ShapeDtypeStruct((e, LW), jnp.float32),
    )(ea, hs, w1, b1.reshape(1, H), w2, b2.reshape(1, H * H))


def _bn_relu_pad(t, gam, bet):
    mu = jnp.mean(t, axis=0, keepdims=True)
    d = t - mu
    var = jnp.mean(d * d, axis=0, keepdims=True)
    o = jnp.maximum(d * lax.rsqrt(var + 1e-5) * gam + bet, 0.0)
    return jnp.concatenate(
        [o, jnp.zeros((t.shape[0], LW - H), jnp.float32)], axis=1)


def _upd1_body(h_ref, p_ref, root_ref, bias_ref, gam_ref, bet_ref,
               o_ref, ci_ref):
    psum = jnp.concatenate(
        [p_ref[0, :_NR], p_ref[1, :_NR]], axis=0)[:h_ref.shape[0]]
    cnt = jnp.maximum(psum[:, H:H + 1], 1.0)
    ci = 1.0 / cnt
    t = (jnp.dot(h_ref[:, :H], root_ref[:],
                 preferred_element_type=jnp.float32)
         + psum[:, :H] * ci + bias_ref[:])
    o_ref[:] = _bn_relu_pad(t, gam_ref[:], bet_ref[:])
    ci_ref[:] = ci


def _upd2_body(h_ref, p_ref, ci_ref, root_ref, bias_ref, gam_ref, bet_ref,
               o_ref):
    psum = jnp.concatenate(
        [p_ref[0, :_NR], p_ref[1, :_NR]], axis=0)[:h_ref.shape[0]]
    t = (jnp.dot(h_ref[:, :H], root_ref[:],
                 preferred_element_type=jnp.float32)
         + psum[:, :H] * ci_ref[:] + bias_ref[:])
    o_ref[:] = _bn_relu_pad(t, gam_ref[:], bet_ref[:])


def _update1(h, part, root, bias, gamma, beta):
    n = h.shape[0]
    return pl.pallas_call(
        _upd1_body,
        out_shape=(jax.ShapeDtypeStruct((n, LW), jnp.float32),
                   jax.ShapeDtypeStruct((n, 1), jnp.float32)),
    )(h, part, root, bias.reshape(1, H), gamma.reshape(1, H),
      beta.reshape(1, H))


def _update2(h, part, ci, root, bias, gamma, beta):
    n = h.shape[0]
    return pl.pallas_call(
        _upd2_body,
        out_shape=jax.ShapeDtypeStruct((n, LW), jnp.float32),
    )(h, part, ci, root, bias.reshape(1, H), gamma.reshape(1, H),
      beta.reshape(1, H))


def _pool_body(h_ref, b_ref, w1_ref, b1_ref, w2_ref, b2_ref, o_ref):
    ids = lax.broadcasted_iota(jnp.int32, (128, 1), 0)
    m = (b_ref[:] == ids).astype(jnp.float32)
    gs = jnp.dot(m, h_ref[:, :H], preferred_element_type=jnp.float32)
    ones = jnp.ones((h_ref.shape[0], 1), jnp.float32)
    gc = jnp.dot(m, ones, preferred_element_type=jnp.float32)
    g = gs / jnp.maximum(gc, 1.0)
    hm = jnp.maximum(
        jnp.dot(g, w1_ref[:], preferred_element_type=jnp.float32)
        + b1_ref[:], 0.0)
    o_ref[:] = (jnp.dot(hm, w2_ref[:], preferred_element_type=jnp.float32)
                + b2_ref[:])


def _pool(h, batch_row, w1, b1, w2, b2):
    hh = w1.shape[1]
    return pl.pallas_call(
        _pool_body,
        out_shape=jax.ShapeDtypeStruct((128, 1), jnp.float32),
    )(h, batch_row, w1, b1.reshape(1, hh), w2, b2.reshape(1, 1))


_CH = 128
_NR = 5120


def _sc_gather(h, idx):
    e = idx.shape[0]
    nch = e // _CH
    nw = 32
    per_w = (nch + nw - 1) // nw
    mesh = plsc.VectorSubcoreMesh(core_axis_name="c", subcore_axis_name="s")

    @functools.partial(
        pl.kernel,
        out_type=jax.ShapeDtypeStruct((e, LW), jnp.float32),
        mesh=mesh,
        scratch_types=[
            pltpu.VMEM((_CH,), jnp.int32),
            pltpu.VMEM((_CH, LW), jnp.float32),
            pltpu.SemaphoreType.DMA,
        ],
    )
    def k(h_hbm, idx_hbm, out_hbm, idx_v, rows_v, sem):
        c = lax.axis_index("c")
        s = lax.axis_index("s")
        wid = s * 2 + c

        def body(j, _):
            cid = wid + nw * j

            @pl.when(cid < nch)
            def _():
                base = pl.multiple_of(cid * _CH, _CH)
                pltpu.sync_copy(idx_hbm.at[pl.ds(base, _CH)], idx_v)
                pltpu.async_copy(h_hbm.at[idx_v], rows_v, sem).wait()
                pltpu.sync_copy(rows_v, out_hbm.at[pl.ds(base, _CH)])
            return 0

        lax.fori_loop(0, per_w, body, 0, unroll=False)

    return k(h, idx)


def _sc_scatter(msg, dst, zrow):
    e = msg.shape[0]
    nch = e // _CH
    ns = 16
    per_s = (nch + ns - 1) // ns
    rows_t = _NR // 16
    mesh = plsc.VectorSubcoreMesh(core_axis_name="c", subcore_axis_name="s")

    @functools.partial(
        pl.kernel,
        out_type=jax.ShapeDtypeStruct((2, _NR + 8, LW), jnp.float32),
        mesh=mesh,
        scratch_types=[
            pltpu.VMEM((_CH,), jnp.int32),
            pltpu.VMEM((_CH, LW), jnp.float32),
            pltpu.VMEM((rows_t, LW), jnp.float32),
            pltpu.VMEM_SHARED((_NR + 8, LW), jnp.float32),
        ],
    )
    def k(msg_hbm, dst_hbm, z_hbm, out_hbm, idx_v, msg_v, buf_v, acc_s):
        c = lax.axis_index("c")
        s = lax.axis_index("s")
        lo = c * _NR
        seg = pl.multiple_of(s * rows_t, 8)
        pltpu.sync_copy(z_hbm, buf_v)
        pltpu.sync_copy(buf_v, acc_s.at[pl.ds(seg, rows_t)])

        @pl.when(s == 0)
        def _():
            pltpu.sync_copy(buf_v.at[pl.ds(0, 8)],
                            acc_s.at[pl.ds(_NR, 8)])
        plsc.subcore_barrier()

        def body(j, _):
            cid = s + ns * j

            @pl.when(cid < nch)
            def _():
                base = pl.multiple_of(cid * _CH, _CH)
                pltpu.sync_copy(dst_hbm.at[pl.ds(base, _CH)], idx_v)
                pltpu.sync_copy(msg_hbm.at[pl.ds(base, _CH)], msg_v)
                for g in range(_CH // 16):
                    v = idx_v[pl.ds(g * 16, 16)]
                    rel = v - lo
                    inr = (rel >= 0) & (rel < _NR)
                    idx_v[pl.ds(g * 16, 16)] = jnp.where(inr, rel, _NR)
                pltpu.sync_copy(msg_v, acc_s.at[idx_v], add=True)
            return 0

        lax.fori_loop(0, per_s, body, 0, unroll=False)
        plsc.subcore_barrier()
        pltpu.sync_copy(acc_s.at[pl.ds(seg, rows_t)], buf_v)
        pltpu.sync_copy(buf_v, out_hbm.at[c, pl.ds(seg, rows_t)])

    return k(msg, dst, zrow)


def kernel(x, edge_index, edge_attr, batch, params):
    n = x.shape[0]
    src = edge_index[0]
    dst = edge_index[1]
    h = _embed(x, params["emb_W"], params["emb_b"])
    ci = None
    zrow = jnp.zeros((_NR // 16, LW), jnp.float32)
    for li, lp in enumerate(params["layers"]):
        hs = _sc_gather(h, src)
        msg = _msg(edge_attr, hs, lp["ew1"], lp["eb1"], lp["ew2"], lp["eb2"])
        part = _sc_scatter(msg, dst, zrow)
        if li == 0:
            h, ci = _update1(h, part, lp["root"], lp["bias"], lp["gamma"],
                             lp["beta"])
        else:
            h = _update2(h, part, ci, lp["root"], lp["bias"], lp["gamma"],
                         lp["beta"])
    npad = 10240
    hp = jnp.pad(h, ((0, npad - n), (0, 0)))
    bp = jnp.pad(batch.astype(jnp.int32), (0, npad - n),
                 constant_values=-1).reshape(1, npad)
    out = _pool(hp, bp, params["ro_W1"], params["ro_b1"], params["ro_W2"],
                params["ro_b2"])
    return out[:64]


# speedup vs baseline: 2.4270x; 2.4270x over previous
import functools
import jax
import jax.numpy as jnp
from jax import lax
from jax.experimental import pallas as pl
from jax.experimental.pallas import tpu as pltpu
from jax.experimental.pallas import tpu_sc as plsc

H = 32
LW = 128


def _embed_body(x_ref, w_ref, b_ref, o_ref):
    h = jnp.maximum(
        jnp.dot(x_ref[:], w_ref[:], preferred_element_type=jnp.float32)
        + b_ref[:], 0.0)
    o_ref[:] = jnp.concatenate(
        [h, jnp.zeros((h.shape[0], LW - H), jnp.float32)], axis=1)


def _embed(x, w, b):
    n = x.shape[0]
    return pl.pallas_call(
        _embed_body,
        out_shape=jax.ShapeDtypeStruct((n, LW), jnp.float32),
    )(x, w, b.reshape(1, H))


def _msg_body(ea_ref, hs_ref, w1_ref, b1_ref, w2r_ref, eb2_ref,
              rrep_ref, rtil_ref, o_ref):
    ev = jnp.maximum(
        jnp.dot(ea_ref[:], w1_ref[:], preferred_element_type=jnp.float32)
        + b1_ref[:], 0.0)
    hs = hs_ref[:, :H]
    hs_rep = jnp.dot(hs, rrep_ref[:], preferred_element_type=jnp.float32)
    ev_til = jnp.dot(ev, rtil_ref[:], preferred_element_type=jnp.float32)
    p = hs_rep * ev_til
    msg = (jnp.dot(p, w2r_ref[:], preferred_element_type=jnp.float32)
           + jnp.dot(hs, eb2_ref[:], preferred_element_type=jnp.float32))
    t = msg.shape[0]
    o_ref[:] = jnp.concatenate(
        [msg, jnp.ones((t, 1), jnp.float32),
         jnp.zeros((t, LW - H - 1), jnp.float32)], axis=1)


def _msg(ea, hs, w1, b1, w2r, eb2, rrep, rtil):
    e = ea.shape[0]
    de = ea.shape[1]
    t = 640
    grid = (e // t,)
    return pl.pallas_call(
        _msg_body,
        grid=grid,
        in_specs=[
            pl.BlockSpec((t, de), lambda i: (i, 0)),
            pl.BlockSpec((t, LW), lambda i: (i, 0)),
            pl.BlockSpec((de, H), lambda i: (0, 0)),
            pl.BlockSpec((1, H), lambda i: (0, 0)),
            pl.BlockSpec((H * H, H), lambda i: (0, 0)),
            pl.BlockSpec((H, H), lambda i: (0, 0)),
            pl.BlockSpec((H, H * H), lambda i: (0, 0)),
            pl.BlockSpec((H, H * H), lambda i: (0, 0)),
        ],
        out_specs=pl.BlockSpec((t, LW), lambda i: (i, 0)),
        out_shape=jax.ShapeDtypeStruct((e, LW), jnp.float32),
    )(ea, hs, w1, b1.reshape(1, H), w2r, eb2, rrep, rtil)


def _bn_relu_pad(t, gam, bet):
    mu = jnp.mean(t, axis=0, keepdims=True)
    d = t - mu
    var = jnp.mean(d * d, axis=0, keepdims=True)
    o = jnp.maximum(d * lax.rsqrt(var + 1e-5) * gam + bet, 0.0)
    return jnp.concatenate(
        [o, jnp.zeros((t.shape[0], LW - H), jnp.float32)], axis=1)


def _upd1_body(h_ref, p_ref, root_ref, bias_ref, gam_ref, bet_ref,
               o_ref, ci_ref):
    psum = jnp.concatenate(
        [p_ref[0, :_NR], p_ref[1, :_NR]], axis=0)[:h_ref.shape[0]]
    cnt = jnp.maximum(psum[:, H:H + 1], 1.0)
    ci = 1.0 / cnt
    t = (jnp.dot(h_ref[:, :H], root_ref[:],
                 preferred_element_type=jnp.float32)
         + psum[:, :H] * ci + bias_ref[:])
    o_ref[:] = _bn_relu_pad(t, gam_ref[:], bet_ref[:])
    ci_ref[:] = ci


def _upd2_body(h_ref, p_ref, ci_ref, root_ref, bias_ref, gam_ref, bet_ref,
               o_ref):
    psum = jnp.concatenate(
        [p_ref[0, :_NR], p_ref[1, :_NR]], axis=0)[:h_ref.shape[0]]
    t = (jnp.dot(h_ref[:, :H], root_ref[:],
                 preferred_element_type=jnp.float32)
         + psum[:, :H] * ci_ref[:] + bias_ref[:])
    o_ref[:] = _bn_relu_pad(t, gam_ref[:], bet_ref[:])


def _update1(h, part, root, bias, gamma, beta):
    n = h.shape[0]
    return pl.pallas_call(
        _upd1_body,
        out_shape=(jax.ShapeDtypeStruct((n, LW), jnp.float32),
                   jax.ShapeDtypeStruct((n, 1), jnp.float32)),
    )(h, part, root, bias.reshape(1, H), gamma.reshape(1, H),
      beta.reshape(1, H))


def _update2(h, part, ci, root, bias, gamma, beta):
    n = h.shape[0]
    return pl.pallas_call(
        _upd2_body,
        out_shape=jax.ShapeDtypeStruct((n, LW), jnp.float32),
    )(h, part, ci, root, bias.reshape(1, H), gamma.reshape(1, H),
      beta.reshape(1, H))


def _pool_body(h_ref, b_ref, w1_ref, b1_ref, w2_ref, b2_ref, o_ref):
    ids = lax.broadcasted_iota(jnp.int32, (128, 1), 0)
    m = (b_ref[:] == ids).astype(jnp.float32)
    gs = jnp.dot(m, h_ref[:, :H], preferred_element_type=jnp.float32)
    ones = jnp.ones((h_ref.shape[0], 1), jnp.float32)
    gc = jnp.dot(m, ones, preferred_element_type=jnp.float32)
    g = gs / jnp.maximum(gc, 1.0)
    hm = jnp.maximum(
        jnp.dot(g, w1_ref[:], preferred_element_type=jnp.float32)
        + b1_ref[:], 0.0)
    o_ref[:] = (jnp.dot(hm, w2_ref[:], preferred_element_type=jnp.float32)
                + b2_ref[:])


def _pool(h, batch_row, w1, b1, w2, b2):
    hh = w1.shape[1]
    return pl.pallas_call(
        _pool_body,
        out_shape=jax.ShapeDtypeStruct((128, 1), jnp.float32),
    )(h, batch_row, w1, b1.reshape(1, hh), w2, b2.reshape(1, 1))


_CH = 128
_NR = 5120


def _sc_gather(h, idx):
    e = idx.shape[0]
    nch = e // _CH
    nw = 32
    per_w = (nch + nw - 1) // nw
    mesh = plsc.VectorSubcoreMesh(core_axis_name="c", subcore_axis_name="s")

    @functools.partial(
        pl.kernel,
        out_type=jax.ShapeDtypeStruct((e, LW), jnp.float32),
        mesh=mesh,
        scratch_types=[
            pltpu.VMEM((_CH,), jnp.int32),
            pltpu.VMEM((_CH, LW), jnp.float32),
            pltpu.SemaphoreType.DMA,
        ],
    )
    def k(h_hbm, idx_hbm, out_hbm, idx_v, rows_v, sem):
        c = lax.axis_index("c")
        s = lax.axis_index("s")
        wid = s * 2 + c

        def body(j, _):
            cid = wid + nw * j

            @pl.when(cid < nch)
            def _():
                base = pl.multiple_of(cid * _CH, _CH)
                pltpu.sync_copy(idx_hbm.at[pl.ds(base, _CH)], idx_v)
                pltpu.async_copy(h_hbm.at[idx_v], rows_v, sem).wait()
                pltpu.sync_copy(rows_v, out_hbm.at[pl.ds(base, _CH)])
            return 0

        lax.fori_loop(0, per_w, body, 0, unroll=False)

    return k(h, idx)


def _sc_scatter(msg, dst, zrow):
    e = msg.shape[0]
    nch = e // _CH
    ns = 16
    per_s = (nch + ns - 1) // ns
    rows_t = _NR // 16
    mesh = plsc.VectorSubcoreMesh(core_axis_name="c", subcore_axis_name="s")

    @functools.partial(
        pl.kernel,
        out_type=jax.ShapeDtypeStruct((2, _NR + 8, LW), jnp.float32),
        mesh=mesh,
        scratch_types=[
            pltpu.VMEM((_CH,), jnp.int32),
            pltpu.VMEM((_CH, LW), jnp.float32),
            pltpu.VMEM((rows_t, LW), jnp.float32),
            pltpu.VMEM_SHARED((_NR + 8, LW), jnp.float32),
        ],
    )
    def k(msg_hbm, dst_hbm, z_hbm, out_hbm, idx_v, msg_v, buf_v, acc_s):
        c = lax.axis_index("c")
        s = lax.axis_index("s")
        lo = c * _NR
        seg = pl.multiple_of(s * rows_t, 8)
        pltpu.sync_copy(z_hbm, buf_v)
        pltpu.sync_copy(buf_v, acc_s.at[pl.ds(seg, rows_t)])

        @pl.when(s == 0)
        def _():
            pltpu.sync_copy(buf_v.at[pl.ds(0, 8)],
                            acc_s.at[pl.ds(_NR, 8)])
        plsc.subcore_barrier()

        def body(j, _):
            cid = s + ns * j

            @pl.when(cid < nch)
            def _():
                base = pl.multiple_of(cid * _CH, _CH)
                pltpu.sync_copy(dst_hbm.at[pl.ds(base, _CH)], idx_v)
                pltpu.sync_copy(msg_hbm.at[pl.ds(base, _CH)], msg_v)
                for g in range(_CH // 16):
                    v = idx_v[pl.ds(g * 16, 16)]
                    rel = v - lo
                    inr = (rel >= 0) & (rel < _NR)
                    idx_v[pl.ds(g * 16, 16)] = jnp.where(inr, rel, _NR)
                pltpu.sync_copy(msg_v, acc_s.at[idx_v], add=True)
            return 0

        lax.fori_loop(0, per_s, body, 0, unroll=False)
        plsc.subcore_barrier()
        pltpu.sync_copy(acc_s.at[pl.ds(seg, rows_t)], buf_v)
        pltpu.sync_copy(buf_v, out_hbm.at[c, pl.ds(seg, rows_t)])

    return k(msg, dst, zrow)


def kernel(x, edge_index, edge_attr, batch, params):
    n = x.shape[0]
    src = edge_index[0]
    dst = edge_index[1]
    h = _embed(x, params["emb_W"], params["emb_b"])
    ci = None
    zrow = jnp.zeros((_NR // 16, LW), jnp.float32)
    eye = jnp.eye(H, dtype=jnp.float32)
    rrep = jnp.repeat(eye, H, axis=1)
    rtil = jnp.tile(eye, (1, H))
    for li, lp in enumerate(params["layers"]):
        w2r = lp["ew2"].reshape(H, H, H).transpose(1, 0, 2).reshape(H * H, H)
        eb2 = lp["eb2"].reshape(H, H)
        hs = _sc_gather(h, src)
        msg = _msg(edge_attr, hs, lp["ew1"], lp["eb1"], w2r, eb2, rrep, rtil)
        part = _sc_scatter(msg, dst, zrow)
        if li == 0:
            h, ci = _update1(h, part, lp["root"], lp["bias"], lp["gamma"],
                             lp["beta"])
        else:
            h = _update2(h, part, ci, lp["root"], lp["bias"], lp["gamma"],
                         lp["beta"])
    npad = 10240
    hp = jnp.pad(h, ((0, npad - n), (0, 0)))
    bp = jnp.pad(batch.astype(jnp.int32), (0, npad - n),
                 constant_values=-1).reshape(1, npad)
    out = _pool(hp, bp, params["ro_W1"], params["ro_b1"], params["ro_W2"],
                params["ro_b2"])
    return out[:64]


# speedup vs baseline: 2.8920x; 1.1916x over previous
import functools
import jax
import jax.numpy as jnp
from jax import lax
from jax.experimental import pallas as pl
from jax.experimental.pallas import tpu as pltpu
from jax.experimental.pallas import tpu_sc as plsc

H = 32
LW = 128


def _embed_body(x_ref, w_ref, b_ref, o_ref):
    h = jnp.maximum(
        jnp.dot(x_ref[:], w_ref[:], preferred_element_type=jnp.float32)
        + b_ref[:], 0.0)
    o_ref[:] = jnp.concatenate(
        [h, jnp.zeros((h.shape[0], LW - H), jnp.float32)], axis=1)


def _embed(x, w, b):
    n = x.shape[0]
    return pl.pallas_call(
        _embed_body,
        out_shape=jax.ShapeDtypeStruct((n, LW), jnp.float32),
    )(x, w, b.reshape(1, H))


def _msg_body(ea_ref, hs_ref, w1_ref, b1_ref, w2r_ref, eb2_ref,
              rrep_ref, rtil_ref, o_ref):
    ev = jnp.maximum(
        jnp.dot(ea_ref[:], w1_ref[:], preferred_element_type=jnp.float32)
        + b1_ref[:], 0.0)
    hs = hs_ref[:, :H]
    hs_rep = jnp.dot(hs, rrep_ref[:], preferred_element_type=jnp.float32)
    ev_til = jnp.dot(ev, rtil_ref[:], preferred_element_type=jnp.float32)
    p = hs_rep * ev_til
    msg = (jnp.dot(p, w2r_ref[:], preferred_element_type=jnp.float32)
           + jnp.dot(hs, eb2_ref[:], preferred_element_type=jnp.float32))
    t = msg.shape[0]
    o_ref[:] = jnp.concatenate(
        [msg, jnp.ones((t, 1), jnp.float32),
         jnp.zeros((t, LW - H - 1), jnp.float32)], axis=1)


def _msg(ea, hs, w1, b1, w2r, eb2, rrep, rtil):
    e = ea.shape[0]
    de = ea.shape[1]
    t = 640
    grid = (e // t,)
    return pl.pallas_call(
        _msg_body,
        grid=grid,
        in_specs=[
            pl.BlockSpec((t, de), lambda i: (i, 0)),
            pl.BlockSpec((t, LW), lambda i: (i, 0)),
            pl.BlockSpec((de, H), lambda i: (0, 0)),
            pl.BlockSpec((1, H), lambda i: (0, 0)),
            pl.BlockSpec((H * H, H), lambda i: (0, 0)),
            pl.BlockSpec((H, H), lambda i: (0, 0)),
            pl.BlockSpec((H, H * H), lambda i: (0, 0)),
            pl.BlockSpec((H, H * H), lambda i: (0, 0)),
        ],
        out_specs=pl.BlockSpec((t, LW), lambda i: (i, 0)),
        out_shape=jax.ShapeDtypeStruct((e, LW), jnp.float32),
    )(ea, hs, w1, b1.reshape(1, H), w2r, eb2, rrep, rtil)


def _bn_relu_pad(t, gam, bet):
    mu = jnp.mean(t, axis=0, keepdims=True)
    d = t - mu
    var = jnp.mean(d * d, axis=0, keepdims=True)
    o = jnp.maximum(d * lax.rsqrt(var + 1e-5) * gam + bet, 0.0)
    return jnp.concatenate(
        [o, jnp.zeros((t.shape[0], LW - H), jnp.float32)], axis=1)


def _upd1_body(h_ref, p_ref, root_ref, bias_ref, gam_ref, bet_ref,
               o_ref, ci_ref):
    psum = jnp.concatenate(
        [p_ref[0, :_NR], p_ref[1, :_NR]], axis=0)[:h_ref.shape[0]]
    cnt = jnp.maximum(psum[:, H:H + 1], 1.0)
    ci = 1.0 / cnt
    t = (jnp.dot(h_ref[:, :H], root_ref[:],
                 preferred_element_type=jnp.float32)
         + psum[:, :H] * ci + bias_ref[:])
    o_ref[:] = _bn_relu_pad(t, gam_ref[:], bet_ref[:])
    ci_ref[:] = ci


def _upd2_body(h_ref, p_ref, ci_ref, root_ref, bias_ref, gam_ref, bet_ref,
               o_ref):
    psum = jnp.concatenate(
        [p_ref[0, :_NR], p_ref[1, :_NR]], axis=0)[:h_ref.shape[0]]
    t = (jnp.dot(h_ref[:, :H], root_ref[:],
                 preferred_element_type=jnp.float32)
         + psum[:, :H] * ci_ref[:] + bias_ref[:])
    o_ref[:] = _bn_relu_pad(t, gam_ref[:], bet_ref[:])


def _update1(h, part, root, bias, gamma, beta):
    n = h.shape[0]
    return pl.pallas_call(
        _upd1_body,
        out_shape=(jax.ShapeDtypeStruct((n, LW), jnp.float32),
                   jax.ShapeDtypeStruct((n, 1), jnp.float32)),
    )(h, part, root, bias.reshape(1, H), gamma.reshape(1, H),
      beta.reshape(1, H))


def _update2(h, part, ci, root, bias, gamma, beta):
    n = h.shape[0]
    return pl.pallas_call(
        _upd2_body,
        out_shape=jax.ShapeDtypeStruct((n, LW), jnp.float32),
    )(h, part, ci, root, bias.reshape(1, H), gamma.reshape(1, H),
      beta.reshape(1, H))


def _pool_body(h_ref, b_ref, w1_ref, b1_ref, w2_ref, b2_ref, o_ref):
    ids = lax.broadcasted_iota(jnp.int32, (128, 1), 0)
    m = (b_ref[:] == ids).astype(jnp.float32)
    gs = jnp.dot(m, h_ref[:, :H], preferred_element_type=jnp.float32)
    ones = jnp.ones((h_ref.shape[0], 1), jnp.float32)
    gc = jnp.dot(m, ones, preferred_element_type=jnp.float32)
    g = gs / jnp.maximum(gc, 1.0)
    hm = jnp.maximum(
        jnp.dot(g, w1_ref[:], preferred_element_type=jnp.float32)
        + b1_ref[:], 0.0)
    o_ref[:] = (jnp.dot(hm, w2_ref[:], preferred_element_type=jnp.float32)
                + b2_ref[:])


def _pool(h, batch_row, w1, b1, w2, b2):
    hh = w1.shape[1]
    return pl.pallas_call(
        _pool_body,
        out_shape=jax.ShapeDtypeStruct((128, 1), jnp.float32),
    )(h, batch_row, w1, b1.reshape(1, hh), w2, b2.reshape(1, 1))


_CH = 128
_NR = 5120


def _sc_gather(h, idx):
    e = idx.shape[0]
    nw = 32
    per_w = e // nw
    nfull = per_w // _CH
    rem = per_w - nfull * _CH
    mesh = plsc.VectorSubcoreMesh(core_axis_name="c", subcore_axis_name="s")

    @functools.partial(
        pl.kernel,
        out_type=jax.ShapeDtypeStruct((e, LW), jnp.float32),
        mesh=mesh,
        scratch_types=[
            pltpu.VMEM((per_w,), jnp.int32),
            pltpu.VMEM((_CH, LW), jnp.float32),
            pltpu.VMEM((_CH, LW), jnp.float32),
            pltpu.SemaphoreType.DMA,
            pltpu.SemaphoreType.DMA,
            pltpu.SemaphoreType.DMA,
            pltpu.SemaphoreType.DMA,
        ],
    )
    def k(h_hbm, idx_hbm, out_hbm, idx_v, rows0, rows1, g0, g1, o0, o1):
        c = lax.axis_index("c")
        s = lax.axis_index("s")
        wid = s * 2 + c
        base_w = pl.multiple_of(wid * per_w, 8)
        rows = (rows0, rows1)
        gsem = (g0, g1)
        osem = (o0, o1)
        pltpu.sync_copy(idx_hbm.at[pl.ds(base_w, per_w)], idx_v)

        def start_g(i, b):
            pltpu.async_copy(h_hbm.at[idx_v.at[pl.ds(i * _CH, _CH)]],
                             rows[b], gsem[b])

        def wait_g(b):
            pltpu.make_async_copy(
                h_hbm.at[idx_v.at[pl.ds(0, _CH)]], rows[b], gsem[b]).wait()

        def start_o(i, b):
            pltpu.async_copy(rows[b],
                             out_hbm.at[pl.ds(base_w + i * _CH, _CH)],
                             osem[b])

        def wait_o(b):
            pltpu.make_async_copy(
                rows[b], out_hbm.at[pl.ds(base_w, _CH)], osem[b]).wait()

        start_g(0, 0)

        def body(jo, _):
            for b in range(2):
                cur = jo * 2 + b

                @pl.when(cur < nfull)
                def _():
                    @pl.when(cur >= 1)
                    def _():
                        wait_o(1 - b)

                    @pl.when(cur + 1 < nfull)
                    def _():
                        start_g(cur + 1, 1 - b)
                    wait_g(b)
                    start_o(cur, b)
            return 0

        lax.fori_loop(0, (nfull + 1) // 2, body, 0, unroll=False)
        wait_o((nfull - 1) % 2)
        if rem:
            rbase = nfull * _CH
            pltpu.async_copy(
                h_hbm.at[idx_v.at[pl.ds(rbase, rem)]],
                rows0.at[pl.ds(0, rem)], g0).wait()
            pltpu.sync_copy(rows0.at[pl.ds(0, rem)],
                            out_hbm.at[pl.ds(base_w + rbase, rem)])

    return k(h, idx)


def _sc_scatter(msg, dst, zrow):
    e = msg.shape[0]
    ns = 16
    per_s = e // ns
    nfull = per_s // _CH
    rem = per_s - nfull * _CH
    rows_t = _NR // 16
    mesh = plsc.VectorSubcoreMesh(core_axis_name="c", subcore_axis_name="s")

    @functools.partial(
        pl.kernel,
        out_type=jax.ShapeDtypeStruct((2, _NR + 8, LW), jnp.float32),
        mesh=mesh,
        scratch_types=[
            pltpu.VMEM((per_s,), jnp.int32),
            pltpu.VMEM((_CH,), jnp.int32),
            pltpu.VMEM((_CH,), jnp.int32),
            pltpu.VMEM((16,), jnp.int32),
            pltpu.VMEM((_CH, LW), jnp.float32),
            pltpu.VMEM((_CH, LW), jnp.float32),
            pltpu.VMEM((rows_t, LW), jnp.float32),
            pltpu.VMEM_SHARED((_NR + 8, LW), jnp.float32),
            pltpu.SemaphoreType.DMA,
            pltpu.SemaphoreType.DMA,
        ],
    )
    def k(msg_hbm, dst_hbm, z_hbm, out_hbm, idx_all, idxb0, idxb1, idxr,
          msg0, msg1, buf_v, acc_s, m0, m1):
        c = lax.axis_index("c")
        s = lax.axis_index("s")
        lo = c * _NR
        base_s = pl.multiple_of(s * per_s, 8)
        seg = pl.multiple_of(s * rows_t, 8)
        msgb = (msg0, msg1)
        idxb = (idxb0, idxb1)
        msem = (m0, m1)
        pltpu.sync_copy(z_hbm, buf_v)
        pltpu.sync_copy(buf_v, acc_s.at[pl.ds(seg, rows_t)])

        @pl.when(s == 0)
        def _():
            pltpu.sync_copy(buf_v.at[pl.ds(0, 8)], acc_s.at[pl.ds(_NR, 8)])
        pltpu.sync_copy(dst_hbm.at[pl.ds(base_s, per_s)], idx_all)
        plsc.subcore_barrier()

        def start_m(i, b):
            pltpu.async_copy(msg_hbm.at[pl.ds(base_s + i * _CH, _CH)],
                             msgb[b], msem[b])

        def wait_m(b):
            pltpu.make_async_copy(
                msg_hbm.at[pl.ds(base_s, _CH)], msgb[b], msem[b]).wait()

        def xform(i, dst_ref, nrow):
            for g in range(nrow // 16):
                v = idx_all[pl.ds(i * _CH + g * 16, 16)]
                rel = v - lo
                inr = (rel >= 0) & (rel < _NR)
                dst_ref[pl.ds(g * 16, 16)] = jnp.where(inr, rel, _NR)

        start_m(0, 0)

        def body(jo, _):
            for b in range(2):
                cur = jo * 2 + b

                @pl.when(cur < nfull)
                def _():
                    @pl.when(cur + 1 < nfull)
                    def _():
                        start_m(cur + 1, 1 - b)
                    xform(cur, idxb[b], _CH)
                    wait_m(b)
                    pltpu.sync_copy(msgb[b], acc_s.at[idxb[b]], add=True)
            return 0

        lax.fori_loop(0, (nfull + 1) // 2, body, 0, unroll=False)
        if rem:
            rbase = nfull * _CH
            xform(nfull, idxr, rem)
            pltpu.async_copy(
                msg_hbm.at[pl.ds(base_s + rbase, rem)],
                msg0.at[pl.ds(0, rem)], m0).wait()
            pltpu.sync_copy(msg0.at[pl.ds(0, rem)], acc_s.at[idxr],
                            add=True)
        plsc.subcore_barrier()
        pltpu.sync_copy(acc_s.at[pl.ds(seg, rows_t)], buf_v)
        pltpu.sync_copy(buf_v, out_hbm.at[c, pl.ds(seg, rows_t)])

    return k(msg, dst, zrow)


def kernel(x, edge_index, edge_attr, batch, params):
    n = x.shape[0]
    src = edge_index[0]
    dst = edge_index[1]
    h = _embed(x, params["emb_W"], params["emb_b"])
    ci = None
    zrow = jnp.zeros((_NR // 16, LW), jnp.float32)
    eye = jnp.eye(H, dtype=jnp.float32)
    rrep = jnp.repeat(eye, H, axis=1)
    rtil = jnp.tile(eye, (1, H))
    for li, lp in enumerate(params["layers"]):
        w2r = lp["ew2"].reshape(H, H, H).transpose(1, 0, 2).reshape(H * H, H)
        eb2 = lp["eb2"].reshape(H, H)
        hs = _sc_gather(h, src)
        msg = _msg(edge_attr, hs, lp["ew1"], lp["eb1"], w2r, eb2, rrep, rtil)
        part = _sc_scatter(msg, dst, zrow)
        if li == 0:
            h, ci = _update1(h, part, lp["root"], lp["bias"], lp["gamma"],
                             lp["beta"])
        else:
            h = _update2(h, part, ci, lp["root"], lp["bias"], lp["gamma"],
                         lp["beta"])
    npad = 10240
    hp = jnp.pad(h, ((0, npad - n), (0, 0)))
    bp = jnp.pad(batch.astype(jnp.int32), (0, npad - n),
                 constant_values=-1).reshape(1, npad)
    out = _pool(hp, bp, params["ro_W1"], params["ro_b1"], params["ro_W2"],
                params["ro_b2"])
    return out[:64]


# speedup vs baseline: 3.2036x; 1.1077x over previous
import functools
import jax
import jax.numpy as jnp
from jax import lax
from jax.experimental import pallas as pl
from jax.experimental.pallas import tpu as pltpu
from jax.experimental.pallas import tpu_sc as plsc

H = 32
LW = 128


def _embed_body(x_ref, w_ref, b_ref, o_ref):
    h = jnp.maximum(
        jnp.dot(x_ref[:], w_ref[:], preferred_element_type=jnp.float32)
        + b_ref[:], 0.0)
    o_ref[:] = jnp.concatenate(
        [h, jnp.zeros((h.shape[0], LW - H), jnp.float32)], axis=1)


def _embed(x, w, b):
    n = x.shape[0]
    return pl.pallas_call(
        _embed_body,
        out_shape=jax.ShapeDtypeStruct((n, LW), jnp.float32),
    )(x, w, b.reshape(1, H))


def _msg_body(ea_ref, hs_ref, w1_ref, b1_ref, w2r_ref, eb2_ref,
              rrep_ref, rtil_ref, o_ref):
    ev = jnp.maximum(
        jnp.dot(ea_ref[:], w1_ref[:], preferred_element_type=jnp.float32)
        + b1_ref[:], 0.0)
    hs = hs_ref[:, :H]
    hs_rep = jnp.dot(hs, rrep_ref[:], preferred_element_type=jnp.float32)
    ev_til = pltpu.repeat(ev, H, axis=1)
    p = hs_rep * ev_til
    msg = (jnp.dot(p, w2r_ref[:], preferred_element_type=jnp.float32)
           + jnp.dot(hs, eb2_ref[:], preferred_element_type=jnp.float32))
    t = msg.shape[0]
    o_ref[:] = jnp.concatenate(
        [msg, jnp.ones((t, 1), jnp.float32),
         jnp.zeros((t, LW - H - 1), jnp.float32)], axis=1)


def _msg(ea, hs, w1, b1, w2r, eb2, rrep, rtil):
    e = ea.shape[0]
    de = ea.shape[1]
    t = 640
    grid = (e // t,)
    return pl.pallas_call(
        _msg_body,
        grid=grid,
        in_specs=[
            pl.BlockSpec((t, de), lambda i: (i, 0)),
            pl.BlockSpec((t, LW), lambda i: (i, 0)),
            pl.BlockSpec((de, H), lambda i: (0, 0)),
            pl.BlockSpec((1, H), lambda i: (0, 0)),
            pl.BlockSpec((H * H, H), lambda i: (0, 0)),
            pl.BlockSpec((H, H), lambda i: (0, 0)),
            pl.BlockSpec((H, H * H), lambda i: (0, 0)),
            pl.BlockSpec((H, H * H), lambda i: (0, 0)),
        ],
        out_specs=pl.BlockSpec((t, LW), lambda i: (i, 0)),
        out_shape=jax.ShapeDtypeStruct((e, LW), jnp.float32),
    )(ea, hs, w1, b1.reshape(1, H), w2r, eb2, rrep, rtil)


def _bn_relu_pad(t, gam, bet):
    mu = jnp.mean(t, axis=0, keepdims=True)
    d = t - mu
    var = jnp.mean(d * d, axis=0, keepdims=True)
    o = jnp.maximum(d * lax.rsqrt(var + 1e-5) * gam + bet, 0.0)
    return jnp.concatenate(
        [o, jnp.zeros((t.shape[0], LW - H), jnp.float32)], axis=1)


def _upd1_body(h_ref, p_ref, root_ref, bias_ref, gam_ref, bet_ref,
               o_ref, ci_ref):
    psum = jnp.concatenate(
        [p_ref[0, :_NR], p_ref[1, :_NR]], axis=0)[:h_ref.shape[0]]
    cnt = jnp.maximum(psum[:, H:H + 1], 1.0)
    ci = 1.0 / cnt
    t = (jnp.dot(h_ref[:, :H], root_ref[:],
                 preferred_element_type=jnp.float32)
         + psum[:, :H] * ci + bias_ref[:])
    o_ref[:] = _bn_relu_pad(t, gam_ref[:], bet_ref[:])
    ci_ref[:] = ci


def _upd2_body(h_ref, p_ref, ci_ref, root_ref, bias_ref, gam_ref, bet_ref,
               o_ref):
    psum = jnp.concatenate(
        [p_ref[0, :_NR], p_ref[1, :_NR]], axis=0)[:h_ref.shape[0]]
    t = (jnp.dot(h_ref[:, :H], root_ref[:],
                 preferred_element_type=jnp.float32)
         + psum[:, :H] * ci_ref[:] + bias_ref[:])
    o_ref[:] = _bn_relu_pad(t, gam_ref[:], bet_ref[:])


def _update1(h, part, root, bias, gamma, beta):
    n = h.shape[0]
    return pl.pallas_call(
        _upd1_body,
        out_shape=(jax.ShapeDtypeStruct((n, LW), jnp.float32),
                   jax.ShapeDtypeStruct((n, 1), jnp.float32)),
    )(h, part, root, bias.reshape(1, H), gamma.reshape(1, H),
      beta.reshape(1, H))


def _update2(h, part, ci, root, bias, gamma, beta):
    n = h.shape[0]
    return pl.pallas_call(
        _upd2_body,
        out_shape=jax.ShapeDtypeStruct((n, LW), jnp.float32),
    )(h, part, ci, root, bias.reshape(1, H), gamma.reshape(1, H),
      beta.reshape(1, H))


def _pool_body(h_ref, b_ref, w1_ref, b1_ref, w2_ref, b2_ref, o_ref):
    ids = lax.broadcasted_iota(jnp.int32, (128, 1), 0)
    m = (b_ref[:] == ids).astype(jnp.float32)
    gs = jnp.dot(m, h_ref[:, :H], preferred_element_type=jnp.float32)
    ones = jnp.ones((h_ref.shape[0], 1), jnp.float32)
    gc = jnp.dot(m, ones, preferred_element_type=jnp.float32)
    g = gs / jnp.maximum(gc, 1.0)
    hm = jnp.maximum(
        jnp.dot(g, w1_ref[:], preferred_element_type=jnp.float32)
        + b1_ref[:], 0.0)
    o_ref[:] = (jnp.dot(hm, w2_ref[:], preferred_element_type=jnp.float32)
                + b2_ref[:])


def _pool(h, batch_row, w1, b1, w2, b2):
    hh = w1.shape[1]
    return pl.pallas_call(
        _pool_body,
        out_shape=jax.ShapeDtypeStruct((128, 1), jnp.float32),
    )(h, batch_row, w1, b1.reshape(1, hh), w2, b2.reshape(1, 1))


_CH = 128
_NR = 5120


def _sc_gather(h, idx):
    e = idx.shape[0]
    nw = 32
    per_w = e // nw
    nfull = per_w // _CH
    rem = per_w - nfull * _CH
    mesh = plsc.VectorSubcoreMesh(core_axis_name="c", subcore_axis_name="s")

    @functools.partial(
        pl.kernel,
        out_type=jax.ShapeDtypeStruct((e, LW), jnp.float32),
        mesh=mesh,
        scratch_types=[
            pltpu.VMEM((per_w,), jnp.int32),
            pltpu.VMEM((_CH, LW), jnp.float32),
            pltpu.VMEM((_CH, LW), jnp.float32),
            pltpu.SemaphoreType.DMA,
            pltpu.SemaphoreType.DMA,
            pltpu.SemaphoreType.DMA,
            pltpu.SemaphoreType.DMA,
        ],
    )
    def k(h_hbm, idx_hbm, out_hbm, idx_v, rows0, rows1, g0, g1, o0, o1):
        c = lax.axis_index("c")
        s = lax.axis_index("s")
        wid = s * 2 + c
        base_w = pl.multiple_of(wid * per_w, 8)
        rows = (rows0, rows1)
        gsem = (g0, g1)
        osem = (o0, o1)
        pltpu.sync_copy(idx_hbm.at[pl.ds(base_w, per_w)], idx_v)

        def start_g(i, b):
            pltpu.async_copy(h_hbm.at[idx_v.at[pl.ds(i * _CH, _CH)]],
                             rows[b], gsem[b])

        def wait_g(b):
            pltpu.make_async_copy(
                h_hbm.at[idx_v.at[pl.ds(0, _CH)]], rows[b], gsem[b]).wait()

        def start_o(i, b):
            pltpu.async_copy(rows[b],
                             out_hbm.at[pl.ds(base_w + i * _CH, _CH)],
                             osem[b])

        def wait_o(b):
            pltpu.make_async_copy(
                rows[b], out_hbm.at[pl.ds(base_w, _CH)], osem[b]).wait()

        start_g(0, 0)

        def body(jo, _):
            for b in range(2):
                cur = jo * 2 + b

                @pl.when(cur < nfull)
                def _():
                    @pl.when(cur >= 1)
                    def _():
                        wait_o(1 - b)

                    @pl.when(cur + 1 < nfull)
                    def _():
                        start_g(cur + 1, 1 - b)
                    wait_g(b)
                    start_o(cur, b)
            return 0

        lax.fori_loop(0, (nfull + 1) // 2, body, 0, unroll=False)
        wait_o((nfull - 1) % 2)
        if rem:
            rbase = nfull * _CH
            pltpu.async_copy(
                h_hbm.at[idx_v.at[pl.ds(rbase, rem)]],
                rows0.at[pl.ds(0, rem)], g0).wait()
            pltpu.sync_copy(rows0.at[pl.ds(0, rem)],
                            out_hbm.at[pl.ds(base_w + rbase, rem)])

    return k(h, idx)


def _sc_scatter(msg, dst, zrow):
    e = msg.shape[0]
    ns = 16
    per_s = e // ns
    nfull = per_s // _CH
    rem = per_s - nfull * _CH
    rows_t = _NR // 16
    mesh = plsc.VectorSubcoreMesh(core_axis_name="c", subcore_axis_name="s")

    @functools.partial(
        pl.kernel,
        out_type=jax.ShapeDtypeStruct((2, _NR + 8, LW), jnp.float32),
        mesh=mesh,
        scratch_types=[
            pltpu.VMEM((per_s,), jnp.int32),
            pltpu.VMEM((_CH,), jnp.int32),
            pltpu.VMEM((_CH,), jnp.int32),
            pltpu.VMEM((16,), jnp.int32),
            pltpu.VMEM((_CH, LW), jnp.float32),
            pltpu.VMEM((_CH, LW), jnp.float32),
            pltpu.VMEM((rows_t, LW), jnp.float32),
            pltpu.VMEM_SHARED((_NR + 8, LW), jnp.float32),
            pltpu.SemaphoreType.DMA,
            pltpu.SemaphoreType.DMA,
        ],
    )
    def k(msg_hbm, dst_hbm, z_hbm, out_hbm, idx_all, idxb0, idxb1, idxr,
          msg0, msg1, buf_v, acc_s, m0, m1):
        c = lax.axis_index("c")
        s = lax.axis_index("s")
        lo = c * _NR
        base_s = pl.multiple_of(s * per_s, 8)
        seg = pl.multiple_of(s * rows_t, 8)
        msgb = (msg0, msg1)
        idxb = (idxb0, idxb1)
        msem = (m0, m1)
        pltpu.sync_copy(z_hbm, buf_v)
        pltpu.sync_copy(buf_v, acc_s.at[pl.ds(seg, rows_t)])

        @pl.when(s == 0)
        def _():
            pltpu.sync_copy(buf_v.at[pl.ds(0, 8)], acc_s.at[pl.ds(_NR, 8)])
        pltpu.sync_copy(dst_hbm.at[pl.ds(base_s, per_s)], idx_all)
        plsc.subcore_barrier()

        def start_m(i, b):
            pltpu.async_copy(msg_hbm.at[pl.ds(base_s + i * _CH, _CH)],
                             msgb[b], msem[b])

        def wait_m(b):
            pltpu.make_async_copy(
                msg_hbm.at[pl.ds(base_s, _CH)], msgb[b], msem[b]).wait()

        def xform(i, dst_ref, nrow):
            for g in range(nrow // 16):
                v = idx_all[pl.ds(i * _CH + g * 16, 16)]
                rel = v - lo
                inr = (rel >= 0) & (rel < _NR)
                dst_ref[pl.ds(g * 16, 16)] = jnp.where(inr, rel, _NR)

        start_m(0, 0)

        def body(jo, _):
            for b in range(2):
                cur = jo * 2 + b

                @pl.when(cur < nfull)
                def _():
                    @pl.when(cur + 1 < nfull)
                    def _():
                        start_m(cur + 1, 1 - b)
                    xform(cur, idxb[b], _CH)
                    wait_m(b)
                    pltpu.sync_copy(msgb[b], acc_s.at[idxb[b]], add=True)
            return 0

        lax.fori_loop(0, (nfull + 1) // 2, body, 0, unroll=False)
        if rem:
            rbase = nfull * _CH
            xform(nfull, idxr, rem)
            pltpu.async_copy(
                msg_hbm.at[pl.ds(base_s + rbase, rem)],
                msg0.at[pl.ds(0, rem)], m0).wait()
            pltpu.sync_copy(msg0.at[pl.ds(0, rem)], acc_s.at[idxr],
                            add=True)
        plsc.subcore_barrier()
        pltpu.sync_copy(acc_s.at[pl.ds(seg, rows_t)], buf_v)
        pltpu.sync_copy(buf_v, out_hbm.at[c, pl.ds(seg, rows_t)])

    return k(msg, dst, zrow)


def kernel(x, edge_index, edge_attr, batch, params):
    n = x.shape[0]
    src = edge_index[0]
    dst = edge_index[1]
    h = _embed(x, params["emb_W"], params["emb_b"])
    ci = None
    zrow = jnp.zeros((_NR // 16, LW), jnp.float32)
    eye = jnp.eye(H, dtype=jnp.float32)
    rrep = jnp.repeat(eye, H, axis=1)
    rtil = jnp.tile(eye, (1, H))
    for li, lp in enumerate(params["layers"]):
        w2r = (lp["ew2"].reshape(H, H, H).transpose(1, 0, 2)
               .reshape(H * H, H))
        eb2 = lp["eb2"].reshape(H, H)
        hs = _sc_gather(h, src)
        msg = _msg(edge_attr, hs, lp["ew1"], lp["eb1"], w2r, eb2, rrep, rtil)
        part = _sc_scatter(msg, dst, zrow)
        if li == 0:
            h, ci = _update1(h, part, lp["root"], lp["bias"], lp["gamma"],
                             lp["beta"])
        else:
            h = _update2(h, part, ci, lp["root"], lp["bias"], lp["gamma"],
                         lp["beta"])
    npad = 10240
    hp = jnp.pad(h, ((0, npad - n), (0, 0)))
    bp = jnp.pad(batch.astype(jnp.int32), (0, npad - n),
                 constant_values=-1).reshape(1, npad)
    out = _pool(hp, bp, params["ro_W1"], params["ro_b1"], params["ro_W2"],
                params["ro_b2"])
    return out[:64]


# speedup vs baseline: 4.3674x; 1.3633x over previous
import functools
import jax
import jax.numpy as jnp
from jax import lax
from jax.experimental import pallas as pl
from jax.experimental.pallas import tpu as pltpu
from jax.experimental.pallas import tpu_sc as plsc

H = 32
LW = 128


def _embed_body(x_ref, w_ref, b_ref, o_ref):
    h = jnp.maximum(
        jnp.dot(x_ref[:], w_ref[:], preferred_element_type=jnp.float32)
        + b_ref[:], 0.0)
    o_ref[:] = jnp.concatenate(
        [h, jnp.zeros((h.shape[0], LW - H), jnp.float32)], axis=1)


def _embed(x, w, b):
    n = x.shape[0]
    return pl.pallas_call(
        _embed_body,
        out_shape=jax.ShapeDtypeStruct((n, LW), jnp.float32),
    )(x, w, b.reshape(1, H))


def _msg_body(ea_ref, hs_ref, w1_ref, b1_ref, w2r_ref, eb2_ref,
              rrep_ref, rtil_ref, o_ref):
    ev = jnp.maximum(
        jnp.dot(ea_ref[:], w1_ref[:], preferred_element_type=jnp.float32)
        + b1_ref[:], 0.0)
    hs = hs_ref[:, :H]
    hs_rep = jnp.dot(hs, rrep_ref[:], preferred_element_type=jnp.float32)
    ev_til = pltpu.repeat(ev, H, axis=1)
    p = hs_rep * ev_til
    msg = (jnp.dot(p, w2r_ref[:], preferred_element_type=jnp.float32)
           + jnp.dot(hs, eb2_ref[:], preferred_element_type=jnp.float32))
    t = msg.shape[0]
    o_ref[:] = jnp.concatenate(
        [msg, jnp.ones((t, 1), jnp.float32),
         jnp.zeros((t, LW - H - 1), jnp.float32)], axis=1)


def _msg(ea, hs, w1, b1, w2r, eb2, rrep, rtil):
    e = ea.shape[0]
    de = ea.shape[1]
    t = 1600
    grid = (e // t,)
    return pl.pallas_call(
        _msg_body,
        grid=grid,
        in_specs=[
            pl.BlockSpec((t, de), lambda i: (i, 0)),
            pl.BlockSpec((t, LW), lambda i: (i, 0)),
            pl.BlockSpec((de, H), lambda i: (0, 0)),
            pl.BlockSpec((1, H), lambda i: (0, 0)),
            pl.BlockSpec((H * H, H), lambda i: (0, 0)),
            pl.BlockSpec((H, H), lambda i: (0, 0)),
            pl.BlockSpec((H, H * H), lambda i: (0, 0)),
            pl.BlockSpec((H, H * H), lambda i: (0, 0)),
        ],
        out_specs=pl.BlockSpec((t, LW), lambda i: (i, 0)),
        out_shape=jax.ShapeDtypeStruct((e, LW), jnp.float32),
    )(ea, hs, w1, b1.reshape(1, H), w2r, eb2, rrep, rtil)


def _bn_relu_pad(t, gam, bet):
    mu = jnp.mean(t, axis=0, keepdims=True)
    d = t - mu
    var = jnp.mean(d * d, axis=0, keepdims=True)
    o = jnp.maximum(d * lax.rsqrt(var + 1e-5) * gam + bet, 0.0)
    return jnp.concatenate(
        [o, jnp.zeros((t.shape[0], LW - H), jnp.float32)], axis=1)


def _psum5(parts, nrow):
    acc = None
    for p_ref in parts:
        v = jnp.concatenate([p_ref[0, :_NR], p_ref[1, :_NR]], axis=0)
        acc = v if acc is None else acc + v
    return acc[:nrow]


def _upd1_body(h_ref, p0, p1, p2, p3, p4, root_ref, bias_ref, gam_ref,
               bet_ref, o_ref, ci_ref):
    psum = _psum5((p0, p1, p2, p3, p4), h_ref.shape[0])
    cnt = jnp.maximum(psum[:, H:H + 1], 1.0)
    ci = 1.0 / cnt
    t = (jnp.dot(h_ref[:, :H], root_ref[:],
                 preferred_element_type=jnp.float32)
         + psum[:, :H] * ci + bias_ref[:])
    o_ref[:] = _bn_relu_pad(t, gam_ref[:], bet_ref[:])
    ci_ref[:] = ci


def _upd2_body(h_ref, p0, p1, p2, p3, p4, ci_ref, root_ref, bias_ref,
               gam_ref, bet_ref, o_ref):
    psum = _psum5((p0, p1, p2, p3, p4), h_ref.shape[0])
    t = (jnp.dot(h_ref[:, :H], root_ref[:],
                 preferred_element_type=jnp.float32)
         + psum[:, :H] * ci_ref[:] + bias_ref[:])
    o_ref[:] = _bn_relu_pad(t, gam_ref[:], bet_ref[:])


def _update1(h, parts, root, bias, gamma, beta):
    n = h.shape[0]
    return pl.pallas_call(
        _upd1_body,
        out_shape=(jax.ShapeDtypeStruct((n, LW), jnp.float32),
                   jax.ShapeDtypeStruct((n, 1), jnp.float32)),
    )(h, *parts, root, bias.reshape(1, H), gamma.reshape(1, H),
      beta.reshape(1, H))


def _update2(h, parts, ci, root, bias, gamma, beta):
    n = h.shape[0]
    return pl.pallas_call(
        _upd2_body,
        out_shape=jax.ShapeDtypeStruct((n, LW), jnp.float32),
    )(h, *parts, ci, root, bias.reshape(1, H), gamma.reshape(1, H),
      beta.reshape(1, H))


def _pool_body(h_ref, b_ref, w1_ref, b1_ref, w2_ref, b2_ref, o_ref):
    ids = lax.broadcasted_iota(jnp.int32, (128, 1), 0)
    m = (b_ref[:] == ids).astype(jnp.float32)
    gs = jnp.dot(m, h_ref[:, :H], preferred_element_type=jnp.float32)
    ones = jnp.ones((h_ref.shape[0], 1), jnp.float32)
    gc = jnp.dot(m, ones, preferred_element_type=jnp.float32)
    g = gs / jnp.maximum(gc, 1.0)
    hm = jnp.maximum(
        jnp.dot(g, w1_ref[:], preferred_element_type=jnp.float32)
        + b1_ref[:], 0.0)
    o_ref[:] = (jnp.dot(hm, w2_ref[:], preferred_element_type=jnp.float32)
                + b2_ref[:])


def _pool(h, batch_row, w1, b1, w2, b2):
    hh = w1.shape[1]
    return pl.pallas_call(
        _pool_body,
        out_shape=jax.ShapeDtypeStruct((128, 1), jnp.float32),
    )(h, batch_row, w1, b1.reshape(1, hh), w2, b2.reshape(1, 1))


_CH = 128
_NR = 5120


def _sc_gather(h, idx):
    e = idx.shape[0]
    nw = 32
    per_w = e // nw
    nfull = per_w // _CH
    rem = per_w - nfull * _CH
    mesh = plsc.VectorSubcoreMesh(core_axis_name="c", subcore_axis_name="s")

    @functools.partial(
        pl.kernel,
        out_type=jax.ShapeDtypeStruct((e, LW), jnp.float32),
        mesh=mesh,
        scratch_types=[
            pltpu.VMEM((per_w,), jnp.int32),
            pltpu.VMEM((_CH, LW), jnp.float32),
            pltpu.VMEM((_CH, LW), jnp.float32),
            pltpu.SemaphoreType.DMA,
            pltpu.SemaphoreType.DMA,
            pltpu.SemaphoreType.DMA,
            pltpu.SemaphoreType.DMA,
        ],
    )
    def k(h_hbm, idx_hbm, out_hbm, idx_v, rows0, rows1, g0, g1, o0, o1):
        c = lax.axis_index("c")
        s = lax.axis_index("s")
        wid = s * 2 + c
        base_w = pl.multiple_of(wid * per_w, 8)
        rows = (rows0, rows1)
        gsem = (g0, g1)
        osem = (o0, o1)
        pltpu.sync_copy(idx_hbm.at[pl.ds(base_w, per_w)], idx_v)

        def start_g(i, b):
            pltpu.async_copy(h_hbm.at[idx_v.at[pl.ds(i * _CH, _CH)]],
                             rows[b], gsem[b])

        def wait_g(b):
            pltpu.make_async_copy(
                h_hbm.at[idx_v.at[pl.ds(0, _CH)]], rows[b], gsem[b]).wait()

        def start_o(i, b):
            pltpu.async_copy(rows[b],
                             out_hbm.at[pl.ds(base_w + i * _CH, _CH)],
                             osem[b])

        def wait_o(b):
            pltpu.make_async_copy(
                rows[b], out_hbm.at[pl.ds(base_w, _CH)], osem[b]).wait()

        start_g(0, 0)

        def body(jo, _):
            for b in range(2):
                cur = jo * 2 + b

                @pl.when(cur < nfull)
                def _():
                    @pl.when(cur >= 1)
                    def _():
                        wait_o(1 - b)

                    @pl.when(cur + 1 < nfull)
                    def _():
                        start_g(cur + 1, 1 - b)
                    wait_g(b)
                    start_o(cur, b)
            return 0

        lax.fori_loop(0, (nfull + 1) // 2, body, 0, unroll=False)
        wait_o((nfull - 1) % 2)
        if rem:
            rbase = nfull * _CH
            pltpu.async_copy(
                h_hbm.at[idx_v.at[pl.ds(rbase, rem)]],
                rows0.at[pl.ds(0, rem)], g0).wait()
            pltpu.sync_copy(rows0.at[pl.ds(0, rem)],
                            out_hbm.at[pl.ds(base_w + rbase, rem)])

    return k(h, idx)


def _sc_scatter(msg, dst, zrow):
    e = msg.shape[0]
    ns = 16
    per_s = e // ns
    nfull = per_s // _CH
    rem = per_s - nfull * _CH
    rows_t = _NR // 16
    mesh = plsc.VectorSubcoreMesh(core_axis_name="c", subcore_axis_name="s")

    @functools.partial(
        pl.kernel,
        out_type=jax.ShapeDtypeStruct((2, _NR + 8, LW), jnp.float32),
        mesh=mesh,
        scratch_types=[
            pltpu.VMEM((per_s,), jnp.int32),
            pltpu.VMEM((_CH,), jnp.int32),
            pltpu.VMEM((_CH,), jnp.int32),
            pltpu.VMEM((max(rem, 16),), jnp.int32),
            pltpu.VMEM((_CH, LW), jnp.float32),
            pltpu.VMEM((_CH, LW), jnp.float32),
            pltpu.VMEM((rows_t, LW), jnp.float32),
            pltpu.VMEM_SHARED((_NR + 8, LW), jnp.float32),
            pltpu.SemaphoreType.DMA,
            pltpu.SemaphoreType.DMA,
        ],
    )
    def k(msg_hbm, dst_hbm, z_hbm, out_hbm, idx_all, idxb0, idxb1, idxr,
          msg0, msg1, buf_v, acc_s, m0, m1):
        c = lax.axis_index("c")
        s = lax.axis_index("s")
        lo = c * _NR
        base_s = pl.multiple_of(s * per_s, 8)
        seg = pl.multiple_of(s * rows_t, 8)
        msgb = (msg0, msg1)
        idxb = (idxb0, idxb1)
        msem = (m0, m1)
        pltpu.sync_copy(z_hbm, buf_v)
        pltpu.sync_copy(buf_v, acc_s.at[pl.ds(seg, rows_t)])

        @pl.when(s == 0)
        def _():
            pltpu.sync_copy(buf_v.at[pl.ds(0, 8)], acc_s.at[pl.ds(_NR, 8)])
        pltpu.sync_copy(dst_hbm.at[pl.ds(base_s, per_s)], idx_all)
        plsc.subcore_barrier()

        def start_m(i, b):
            pltpu.async_copy(msg_hbm.at[pl.ds(base_s + i * _CH, _CH)],
                             msgb[b], msem[b])

        def wait_m(b):
            pltpu.make_async_copy(
                msg_hbm.at[pl.ds(base_s, _CH)], msgb[b], msem[b]).wait()

        def xform(i, dst_ref, nrow):
            for g in range(nrow // 16):
                v = idx_all[pl.ds(i * _CH + g * 16, 16)]
                rel = v - lo
                inr = (rel >= 0) & (rel < _NR)
                dst_ref[pl.ds(g * 16, 16)] = jnp.where(inr, rel, _NR)

        start_m(0, 0)

        def body(jo, _):
            for b in range(2):
                cur = jo * 2 + b

                @pl.when(cur < nfull)
                def _():
                    @pl.when(cur + 1 < nfull)
                    def _():
                        start_m(cur + 1, 1 - b)
                    xform(cur, idxb[b], _CH)
                    wait_m(b)
                    pltpu.sync_copy(msgb[b], acc_s.at[idxb[b]], add=True)
            return 0

        lax.fori_loop(0, (nfull + 1) // 2, body, 0, unroll=False)
        if rem:
            rbase = nfull * _CH
            xform(nfull, idxr, rem)
            pltpu.async_copy(
                msg_hbm.at[pl.ds(base_s + rbase, rem)],
                msg0.at[pl.ds(0, rem)], m0).wait()
            pltpu.sync_copy(msg0.at[pl.ds(0, rem)], acc_s.at[idxr],
                            add=True)
        plsc.subcore_barrier()
        pltpu.sync_copy(acc_s.at[pl.ds(seg, rows_t)], buf_v)
        pltpu.sync_copy(buf_v, out_hbm.at[c, pl.ds(seg, rows_t)])

    return k(msg, dst, zrow)


def kernel(x, edge_index, edge_attr, batch, params):
    n = x.shape[0]
    e = edge_index.shape[1]
    nb = 5
    eb = e // nb
    src = edge_index[0]
    dst = edge_index[1]
    srcs = [lax.slice_in_dim(src, b * eb, (b + 1) * eb) for b in range(nb)]
    dsts = [lax.slice_in_dim(dst, b * eb, (b + 1) * eb) for b in range(nb)]
    eas = [lax.slice_in_dim(edge_attr, b * eb, (b + 1) * eb)
           for b in range(nb)]
    h = _embed(x, params["emb_W"], params["emb_b"])
    ci = None
    zrow = jnp.zeros((_NR // 16, LW), jnp.float32)
    eye = jnp.eye(H, dtype=jnp.float32)
    rrep = jnp.repeat(eye, H, axis=1)
    rtil = jnp.tile(eye, (1, H))
    for li, lp in enumerate(params["layers"]):
        w2r = (lp["ew2"].reshape(H, H, H).transpose(1, 0, 2)
               .reshape(H * H, H))
        eb2 = lp["eb2"].reshape(H, H)
        parts = []
        for b in range(nb):
            hs = _sc_gather(h, srcs[b])
            msg = _msg(eas[b], hs, lp["ew1"], lp["eb1"], w2r, eb2,
                       rrep, rtil)
            parts.append(_sc_scatter(msg, dsts[b], zrow))
        if li == 0:
            h, ci = _update1(h, parts, lp["root"], lp["bias"], lp["gamma"],
                             lp["beta"])
        else:
            h = _update2(h, parts, ci, lp["root"], lp["bias"], lp["gamma"],
                         lp["beta"])
    npad = 10240
    hp = jnp.pad(h, ((0, npad - n), (0, 0)))
    bp = jnp.pad(batch.astype(jnp.int32), (0, npad - n),
                 constant_values=-1).reshape(1, npad)
    out = _pool(hp, bp, params["ro_W1"], params["ro_b1"], params["ro_W2"],
                params["ro_b2"])
    return out[:64]
